# baseline (speedup 1.0000x reference)
import functools

import jax
import jax.numpy as jnp
import numpy as np
from jax import lax
from jax.experimental import pallas as pl
from jax.experimental.pallas import tpu as pltpu
from jax.experimental.pallas import tpu_sc as plsc

NTOKEN = 10000
NINP = 600
NHID = 600
NB = 6
BS = NHID // NB
TOPK = 4
ATT_OUT = BS * 4
DK = 64
MH = 4
MDK = 32
SEQ = 70
BATCH = 64

DPAD = 640
NW = 32
NTOT = SEQ * BATCH
BPAD = 4608
BPW = BPAD // NW


def _sc_gather(table_pad, idx_pad):
  mesh = plsc.VectorSubcoreMesh(core_axis_name="c", subcore_axis_name="s",
                                num_cores=2)

  @functools.partial(
      pl.kernel,
      mesh=mesh,
      out_type=jax.ShapeDtypeStruct((BPAD, DPAD), jnp.float32),
      scratch_types=[
          pltpu.VMEM((BPW,), jnp.int32),
          pltpu.VMEM((BPW, DPAD), jnp.float32),
          pltpu.SemaphoreType.DMA,
      ],
  )
  def k(table_hbm, idx_hbm, out_hbm, idx_v, rows_v, sem):
    wid = lax.axis_index("s") * 2 + lax.axis_index("c")
    base = wid * BPW
    pltpu.sync_copy(idx_hbm.at[pl.ds(base, BPW)], idx_v)
    pltpu.async_copy(table_hbm.at[idx_v], rows_v, sem).wait()
    pltpu.sync_copy(rows_v, out_hbm.at[pl.ds(base, BPW)])

  return k(table_pad, idx_pad)


_F32 = jnp.float32


def _head_seg(rows, cols, transpose=False):
  di = lax.broadcasted_iota(jnp.int32, (rows, cols), 1 if transpose else 0)
  hi = lax.broadcasted_iota(jnp.int32, (rows, cols), 0 if transpose else 1)
  return (di // MDK == hi).astype(_F32)


def _rbf(x):
  return x.astype(jnp.bfloat16).astype(_F32)


def _bdot(a, b):
  return jnp.dot(a.astype(jnp.bfloat16), b.astype(jnp.bfloat16),
                 preferred_element_type=_F32)


def _bdot_t(a, b):
  return lax.dot_general(a.astype(jnp.bfloat16), b.astype(jnp.bfloat16),
                         (((1,), (1,)), ((), ())),
                         preferred_element_type=_F32)


def _core_kernel(hx_ref, cx_ref, iu_ref, wih_ref, whh_ref, lb_ref,
                 mqw_ref, mqb_ref, mkw_ref, mkb_ref, mvw_ref, mvb_ref,
                 hn_ref, cn_ref, q2_ref, k2_ref, v2_ref):
  for n in range(NB):
    sl = slice(n * BS, (n + 1) * BS)
    h_n = hx_ref[:, sl]
    c_n = cx_ref[:, sl]
    iu_n = iu_ref[:, n * ATT_OUT:(n + 1) * ATT_OUT]
    gates = (_bdot_t(h_n, whh_ref[n]) + _bdot_t(iu_n, wih_ref[n])
             + lb_ref[n])
    gi = gates[:, 0:BS]
    gf = gates[:, BS:2 * BS]
    gg = gates[:, 2 * BS:3 * BS]
    go = gates[:, 3 * BS:4 * BS]
    c_new = jax.nn.sigmoid(gf) * c_n + jax.nn.sigmoid(gi) * jnp.tanh(gg)
    h_new = jax.nn.sigmoid(go) * jnp.tanh(c_new)
    hn_ref[:, sl] = h_new
    cn_ref[:, sl] = c_new
    sl2 = slice(n * MH * MDK, (n + 1) * MH * MDK)
    q2_ref[:, sl2] = _bdot(h_new, mqw_ref[...]) + mqb_ref[...]
    k2_ref[:, sl2] = _bdot(h_new, mkw_ref[...]) + mkb_ref[...]
    v2_ref[:, sl2] = _bdot(h_new, mvw_ref[...]) + mvb_ref[...]


def _core_call(hx, cx, iu2d, lstm_wih, lstm_whh, lstm_b3, mq_w, mq_b2,
               mk_w, mk_b2, mv_w, mv_b2):
  shp = jax.ShapeDtypeStruct((BATCH, NHID), jnp.float32)
  shp2 = jax.ShapeDtypeStruct((BATCH, NB * MH * MDK), jnp.float32)
  return pl.pallas_call(
      _core_kernel,
      out_shape=[shp, shp, shp2, shp2, shp2],
  )(hx, cx, iu2d, lstm_wih, lstm_whh, lstm_b3, mq_w, mq_b2,
    mk_w, mk_b2, mv_w, mv_b2)


def _recurrence(x_r, h0, c0, iq_w, iq_b, ik_w, ik_b, iv_w, iv_b,
                lstm_wih, lstm_whh, lstm_b, mq_w, mq_b, mk_w, mk_b,
                mv_w, mv_b, mf_w, mf_b, mg_w, mg_b):
  b = BATCH
  lstm_b3 = lstm_b.reshape(NB, 1, 4 * BS)
  mq_b2 = mq_b.reshape(1, MH * MDK)
  mk_b2 = mk_b.reshape(1, MH * MDK)
  mv_b2 = mv_b.reshape(1, MH * MDK)
  mf_b2 = mf_b.reshape(1, BS)
  mg_b2 = mg_b.reshape(1, BS)

  def step(carry, xt):
    hx, cx = carry
    inp_blocks = jnp.stack([jnp.zeros_like(xt), xt], axis=1)
    q = hx.reshape(b, NB, BS) @ iq_w + iq_b
    k = inp_blocks @ ik_w + ik_b
    v = inp_blocks @ iv_w + iv_b
    scores = jnp.einsum('bnd,bmd->bnm', q, k) / float(np.sqrt(DK))
    iatt = jax.nn.softmax(scores, axis=-1)
    inp_use = jnp.einsum('bnm,bmv->bnv', iatt, v)
    null_att = iatt[:, :, 0]
    _, bottomk = jax.lax.top_k(null_att, NB - TOPK)
    mask = jnp.ones((b, NB), jnp.float32).at[
        jnp.arange(b)[:, None], bottomk].set(0.0)
    hn, cn, q2c, k2c, v2c = _core_call(
        hx, cx, inp_use.reshape(b, NB * ATT_OUT), lstm_wih, lstm_whh,
        lstm_b3, mq_w, mq_b2, mk_w, mk_b2, mv_w, mv_b2)
    q2 = q2c.reshape(b, NB, MH, MDK)
    k2 = k2c.reshape(b, NB, MH, MDK)
    v2 = v2c.reshape(b, NB, MH, MDK)
    s2 = jnp.einsum('bnhd,bmhd->bhnm', q2, k2) / float(np.sqrt(MDK))
    a2 = jax.nn.softmax(s2, axis=-1)
    o2 = jnp.einsum('bhnm,bmhd->bnhd', a2, v2).reshape(b, NB, MH * MDK)
    att_update = (jax.nn.sigmoid(o2 @ mg_w + mg_b)
                  * jnp.tanh(o2 @ mf_w + mf_b))
    h_new = hn.reshape(b, NB, BS)
    c_new = cn.reshape(b, NB, BS)
    h_new2 = h_new + att_update
    mb = mask[:, :, None]
    h_b = hx.reshape(b, NB, BS)
    c_b = cx.reshape(b, NB, BS)
    hx_out = (mb * h_new2 + (1.0 - mb) * h_b).reshape(b, NHID)
    cx_out = (mb * c_new + (1.0 - mb) * c_b).reshape(b, NHID)
    return (hx_out, cx_out), hx_out

  (_, _), outs = jax.lax.scan(step, (h0, c0), x_r, unroll=7)
  return outs


_MT = 448
_NT = 2048


def _decode_kernel(x_ref, w_ref, b_ref, out_ref):
  x = x_ref[...].astype(jnp.bfloat16)
  w = w_ref[...].astype(jnp.bfloat16)
  acc = jnp.dot(x, w, preferred_element_type=jnp.float32)
  out_ref[...] = acc + b_ref[...]


def _decode(outs2d, dec_w, dec_b):
  grid = (pl.cdiv(NTOKEN, _NT), NTOT // _MT)
  return pl.pallas_call(
      _decode_kernel,
      grid=grid,
      in_specs=[
          pl.BlockSpec((_MT, NHID), lambda n, m: (m, 0)),
          pl.BlockSpec((NHID, _NT), lambda n, m: (0, n)),
          pl.BlockSpec((1, _NT), lambda n, m: (0, n)),
      ],
      out_specs=pl.BlockSpec((_MT, _NT), lambda n, m: (m, n)),
      out_shape=jax.ShapeDtypeStruct((NTOT, NTOKEN), jnp.float32),
  )(outs2d, dec_w, dec_b.reshape(1, NTOKEN))


def kernel(input, hidden_h0, hidden_c0, emb, iq_w, iq_b, ik_w, ik_b, iv_w,
           iv_b, lstm_wih, lstm_whh, lstm_b, mq_w, mq_b, mk_w, mk_b, mv_w,
           mv_b, mf_w, mf_b, mg_w, mg_b, dec_w, dec_b):
  table_pad = jnp.pad(emb, ((0, 0), (0, DPAD - NINP)))
  idx = input.reshape(-1).astype(jnp.int32)
  idx_pad = jnp.pad(idx, (0, BPAD - NTOT))
  x_rows = _sc_gather(table_pad, idx_pad)
  x_r = x_rows[:NTOT, :NINP].reshape(SEQ, BATCH, NINP)

  outs = _recurrence(x_r, hidden_h0, hidden_c0, iq_w, iq_b, ik_w, ik_b,
                     iv_w, iv_b, lstm_wih, lstm_whh, lstm_b, mq_w, mq_b,
                     mk_w, mk_b, mv_w, mv_b, mf_w, mf_b, mg_w, mg_b)

  logits = _decode(outs.reshape(NTOT, NHID), dec_w, dec_b)
  return logits.reshape(SEQ, BATCH, NTOKEN)


# speedup vs baseline: 1.1342x; 1.1342x over previous
import functools

import jax
import jax.numpy as jnp
import numpy as np
from jax import lax
from jax.experimental import pallas as pl
from jax.experimental.pallas import tpu as pltpu
from jax.experimental.pallas import tpu_sc as plsc

NTOKEN = 10000
NINP = 600
NHID = 600
NB = 6
BS = NHID // NB
TOPK = 4
ATT_OUT = BS * 4
DK = 64
MH = 4
MDK = 32
SEQ = 70
BATCH = 64

DPAD = 640
NW = 32
NTOT = SEQ * BATCH
BPAD = 4608
BPW = BPAD // NW


def _sc_gather(table_pad, idx_pad):
  mesh = plsc.VectorSubcoreMesh(core_axis_name="c", subcore_axis_name="s",
                                num_cores=2)

  @functools.partial(
      pl.kernel,
      mesh=mesh,
      out_type=jax.ShapeDtypeStruct((BPAD, DPAD), jnp.float32),
      scratch_types=[
          pltpu.VMEM((BPW,), jnp.int32),
          pltpu.VMEM((BPW, DPAD), jnp.float32),
          pltpu.SemaphoreType.DMA,
      ],
  )
  def k(table_hbm, idx_hbm, out_hbm, idx_v, rows_v, sem):
    wid = lax.axis_index("s") * 2 + lax.axis_index("c")
    base = wid * BPW
    pltpu.sync_copy(idx_hbm.at[pl.ds(base, BPW)], idx_v)
    pltpu.async_copy(table_hbm.at[idx_v], rows_v, sem).wait()
    pltpu.sync_copy(rows_v, out_hbm.at[pl.ds(base, BPW)])

  return k(table_pad, idx_pad)


_F32 = jnp.float32


def _head_seg(rows, cols, transpose=False):
  di = lax.broadcasted_iota(jnp.int32, (rows, cols), 1 if transpose else 0)
  hi = lax.broadcasted_iota(jnp.int32, (rows, cols), 0 if transpose else 1)
  return (di // MDK == hi).astype(_F32)


def _rbf(x):
  return x.astype(jnp.bfloat16).astype(_F32)


def _bdot(a, b):
  return jnp.dot(a.astype(jnp.bfloat16), b.astype(jnp.bfloat16),
                 preferred_element_type=_F32)


def _bdot_t(a, b):
  return lax.dot_general(a.astype(jnp.bfloat16), b.astype(jnp.bfloat16),
                         (((1,), (1,)), ((), ())),
                         preferred_element_type=_F32)


def _core_kernel(hx_ref, cx_ref, iu_ref, wih_ref, whh_ref, lb_ref,
                 mqw_ref, mqb_ref, mkw_ref, mkb_ref, mvw_ref, mvb_ref,
                 hn_ref, cn_ref, q2_ref, k2_ref, v2_ref):
  for n in range(NB):
    sl = slice(n * BS, (n + 1) * BS)
    h_n = hx_ref[:, sl]
    c_n = cx_ref[:, sl]
    iu_n = iu_ref[:, n * ATT_OUT:(n + 1) * ATT_OUT]
    gates = (_bdot_t(h_n, whh_ref[n]) + _bdot_t(iu_n, wih_ref[n])
             + lb_ref[n])
    gi = gates[:, 0:BS]
    gf = gates[:, BS:2 * BS]
    gg = gates[:, 2 * BS:3 * BS]
    go = gates[:, 3 * BS:4 * BS]
    c_new = jax.nn.sigmoid(gf) * c_n + jax.nn.sigmoid(gi) * jnp.tanh(gg)
    h_new = jax.nn.sigmoid(go) * jnp.tanh(c_new)
    hn_ref[:, sl] = h_new
    cn_ref[:, sl] = c_new
    sl2 = slice(n * MH * MDK, (n + 1) * MH * MDK)
    q2_ref[:, sl2] = _bdot(h_new, mqw_ref[...]) + mqb_ref[...]
    k2_ref[:, sl2] = _bdot(h_new, mkw_ref[...]) + mkb_ref[...]
    v2_ref[:, sl2] = _bdot(h_new, mvw_ref[...]) + mvb_ref[...]


def _core_call(hx, cx, iu2d, lstm_wih, lstm_whh, lstm_b3, mq_w, mq_b2,
               mk_w, mk_b2, mv_w, mv_b2):
  shp = jax.ShapeDtypeStruct((BATCH, NHID), jnp.float32)
  shp2 = jax.ShapeDtypeStruct((BATCH, NB * MH * MDK), jnp.float32)
  return pl.pallas_call(
      _core_kernel,
      out_shape=[shp, shp, shp2, shp2, shp2],
  )(hx, cx, iu2d, lstm_wih, lstm_whh, lstm_b3, mq_w, mq_b2,
    mk_w, mk_b2, mv_w, mv_b2)


def _recurrence(x_r, h0, c0, iq_w, iq_b, ik_w, ik_b, iv_w, iv_b,
                lstm_wih, lstm_whh, lstm_b, mq_w, mq_b, mk_w, mk_b,
                mv_w, mv_b, mf_w, mf_b, mg_w, mg_b):
  b = BATCH
  lstm_b3 = lstm_b.reshape(NB, 1, 4 * BS)
  mq_b2 = mq_b.reshape(1, MH * MDK)
  mk_b2 = mk_b.reshape(1, MH * MDK)
  mv_b2 = mv_b.reshape(1, MH * MDK)
  mf_b2 = mf_b.reshape(1, BS)
  mg_b2 = mg_b.reshape(1, BS)

  def step(carry, xt):
    hx, cx = carry
    inp_blocks = jnp.stack([jnp.zeros_like(xt), xt], axis=1)
    q = hx.reshape(b, NB, BS) @ iq_w + iq_b
    k = inp_blocks @ ik_w + ik_b
    v = inp_blocks @ iv_w + iv_b
    scores = jnp.einsum('bnd,bmd->bnm', q, k) / float(np.sqrt(DK))
    iatt = jax.nn.softmax(scores, axis=-1)
    inp_use = jnp.einsum('bnm,bmv->bnv', iatt, v)
    null_att = iatt[:, :, 0]
    _, bottomk = jax.lax.top_k(null_att, NB - TOPK)
    mask = jnp.ones((b, NB), jnp.float32).at[
        jnp.arange(b)[:, None], bottomk].set(0.0)
    hn, cn, q2c, k2c, v2c = _core_call(
        hx, cx, inp_use.reshape(b, NB * ATT_OUT), lstm_wih, lstm_whh,
        lstm_b3, mq_w, mq_b2, mk_w, mk_b2, mv_w, mv_b2)
    q2 = q2c.reshape(b, NB, MH, MDK)
    k2 = k2c.reshape(b, NB, MH, MDK)
    v2 = v2c.reshape(b, NB, MH, MDK)
    s2 = jnp.einsum('bnhd,bmhd->bhnm', q2, k2) / float(np.sqrt(MDK))
    a2 = jax.nn.softmax(s2, axis=-1)
    o2 = jnp.einsum('bhnm,bmhd->bnhd', a2, v2).reshape(b, NB, MH * MDK)
    att_update = (jax.nn.sigmoid(o2 @ mg_w + mg_b)
                  * jnp.tanh(o2 @ mf_w + mf_b))
    h_new = hn.reshape(b, NB, BS)
    c_new = cn.reshape(b, NB, BS)
    h_new2 = h_new + att_update
    mb = mask[:, :, None]
    h_b = hx.reshape(b, NB, BS)
    c_b = cx.reshape(b, NB, BS)
    hx_out = (mb * h_new2 + (1.0 - mb) * h_b).reshape(b, NHID)
    cx_out = (mb * c_new + (1.0 - mb) * c_b).reshape(b, NHID)
    return (hx_out, cx_out), hx_out

  (_, _), outs = jax.lax.scan(step, (h0, c0), x_r)
  return outs


_MT = 448
_NT = 2048


def _decode_kernel(x_ref, w_ref, b_ref, out_ref):
  x = x_ref[...].astype(jnp.bfloat16)
  w = w_ref[...].astype(jnp.bfloat16)
  acc = jnp.dot(x, w, preferred_element_type=jnp.float32)
  out_ref[...] = acc + b_ref[...]


def _decode(outs2d, dec_w, dec_b):
  grid = (pl.cdiv(NTOKEN, _NT), NTOT // _MT)
  return pl.pallas_call(
      _decode_kernel,
      grid=grid,
      in_specs=[
          pl.BlockSpec((_MT, NHID), lambda n, m: (m, 0)),
          pl.BlockSpec((NHID, _NT), lambda n, m: (0, n)),
          pl.BlockSpec((1, _NT), lambda n, m: (0, n)),
      ],
      out_specs=pl.BlockSpec((_MT, _NT), lambda n, m: (m, n)),
      out_shape=jax.ShapeDtypeStruct((NTOT, NTOKEN), jnp.float32),
  )(outs2d, dec_w, dec_b.reshape(1, NTOKEN))


def kernel(input, hidden_h0, hidden_c0, emb, iq_w, iq_b, ik_w, ik_b, iv_w,
           iv_b, lstm_wih, lstm_whh, lstm_b, mq_w, mq_b, mk_w, mk_b, mv_w,
           mv_b, mf_w, mf_b, mg_w, mg_b, dec_w, dec_b):
  table_pad = jnp.pad(emb, ((0, 0), (0, DPAD - NINP)))
  idx = input.reshape(-1).astype(jnp.int32)
  idx_pad = jnp.pad(idx, (0, BPAD - NTOT))
  x_rows = _sc_gather(table_pad, idx_pad)
  x_r = x_rows[:NTOT, :NINP].reshape(SEQ, BATCH, NINP)

  outs = _recurrence(x_r, hidden_h0, hidden_c0, iq_w, iq_b, ik_w, ik_b,
                     iv_w, iv_b, lstm_wih, lstm_whh, lstm_b, mq_w, mq_b,
                     mk_w, mk_b, mv_w, mv_b, mf_w, mf_b, mg_w, mg_b)

  logits = _decode(outs.reshape(NTOT, NHID), dec_w, dec_b)
  return logits.reshape(SEQ, BATCH, NTOKEN)


# speedup vs baseline: 1.2024x; 1.0601x over previous
import functools

import jax
import jax.numpy as jnp
import numpy as np
from jax import lax
from jax.experimental import pallas as pl
from jax.experimental.pallas import tpu as pltpu
from jax.experimental.pallas import tpu_sc as plsc

NTOKEN = 10000
NINP = 600
NHID = 600
NB = 6
BS = NHID // NB
TOPK = 4
ATT_OUT = BS * 4
DK = 64
MH = 4
MDK = 32
SEQ = 70
BATCH = 64

DPAD = 640
NW = 32
NTOT = SEQ * BATCH
BPAD = 4608
BPW = BPAD // NW


def _sc_gather(table_pad, idx_pad):
  mesh = plsc.VectorSubcoreMesh(core_axis_name="c", subcore_axis_name="s",
                                num_cores=2)

  @functools.partial(
      pl.kernel,
      mesh=mesh,
      out_type=jax.ShapeDtypeStruct((BPAD, DPAD), jnp.float32),
      scratch_types=[
          pltpu.VMEM((BPW,), jnp.int32),
          pltpu.VMEM((BPW, DPAD), jnp.float32),
          pltpu.SemaphoreType.DMA,
      ],
  )
  def k(table_hbm, idx_hbm, out_hbm, idx_v, rows_v, sem):
    wid = lax.axis_index("s") * 2 + lax.axis_index("c")
    base = wid * BPW
    pltpu.sync_copy(idx_hbm.at[pl.ds(base, BPW)], idx_v)
    pltpu.async_copy(table_hbm.at[idx_v], rows_v, sem).wait()
    pltpu.sync_copy(rows_v, out_hbm.at[pl.ds(base, BPW)])

  return k(table_pad, idx_pad)


_F32 = jnp.float32


def _bdot(a, b):
  return jnp.dot(a.astype(jnp.bfloat16), b.astype(jnp.bfloat16),
                 preferred_element_type=_F32)


def _bdot_t(a, b):
  return lax.dot_general(a.astype(jnp.bfloat16), b.astype(jnp.bfloat16),
                         (((1,), (1,)), ((), ())),
                         preferred_element_type=_F32)


def _core_kernel(hx_ref, cx_ref, iu_ref, wih_ref, whh_ref, lb_ref,
                 mqkvw_ref, mqkvb_ref, out_ref):
  for n in range(NB):
    sl = slice(n * BS, (n + 1) * BS)
    h_n = hx_ref[:, sl]
    c_n = cx_ref[:, sl]
    iu_n = iu_ref[:, n * ATT_OUT:(n + 1) * ATT_OUT]
    gates = (_bdot_t(h_n, whh_ref[n]) + _bdot_t(iu_n, wih_ref[n])
             + lb_ref[n])
    gi = gates[:, 0:BS]
    gf = gates[:, BS:2 * BS]
    gg = gates[:, 2 * BS:3 * BS]
    go = gates[:, 3 * BS:4 * BS]
    c_new = jax.nn.sigmoid(gf) * c_n + jax.nn.sigmoid(gi) * jnp.tanh(gg)
    h_new = jax.nn.sigmoid(go) * jnp.tanh(c_new)
    out_ref[:, sl] = h_new
    out_ref[:, NHID + n * BS:NHID + (n + 1) * BS] = c_new
    base = 2 * NHID + n * 3 * MH * MDK
    out_ref[:, base:base + 3 * MH * MDK] = (
        _bdot(h_new, mqkvw_ref[...]) + mqkvb_ref[...])


def _core_call(hx, cx, iu2d, lstm_wih, lstm_whh, lstm_b3, mqkv_w, mqkv_b):
  width = 2 * NHID + NB * 3 * MH * MDK
  return pl.pallas_call(
      _core_kernel,
      out_shape=jax.ShapeDtypeStruct((BATCH, width), jnp.float32),
  )(hx, cx, iu2d, lstm_wih, lstm_whh, lstm_b3, mqkv_w, mqkv_b)


def _recurrence(x_r, h0, c0, iq_w, iq_b, ik_w, ik_b, iv_w, iv_b,
                lstm_wih, lstm_whh, lstm_b, mq_w, mq_b, mk_w, mk_b,
                mv_w, mv_b, mf_w, mf_b, mg_w, mg_b):
  b = BATCH
  lstm_b3 = lstm_b.reshape(NB, 1, 4 * BS)
  mqkv_w = jnp.concatenate([mq_w, mk_w, mv_w], axis=1)
  mqkv_b = jnp.concatenate([mq_b, mk_b, mv_b]).reshape(1, 3 * MH * MDK)

  def step(carry, xt):
    hx, cx = carry
    inp_blocks = jnp.stack([jnp.zeros_like(xt), xt], axis=1)
    q = hx.reshape(b, NB, BS) @ iq_w + iq_b
    k = inp_blocks @ ik_w + ik_b
    v = inp_blocks @ iv_w + iv_b
    scores = jnp.einsum('bnd,bmd->bnm', q, k) / float(np.sqrt(DK))
    iatt = jax.nn.softmax(scores, axis=-1)
    inp_use = jnp.einsum('bnm,bmv->bnv', iatt, v)
    null_att = iatt[:, :, 0]
    _, bottomk = jax.lax.top_k(null_att, NB - TOPK)
    mask = jnp.ones((b, NB), jnp.float32).at[
        jnp.arange(b)[:, None], bottomk].set(0.0)
    core = _core_call(
        hx, cx, inp_use.reshape(b, NB * ATT_OUT), lstm_wih, lstm_whh,
        lstm_b3, mqkv_w, mqkv_b)
    hn = core[:, :NHID]
    cn = core[:, NHID:2 * NHID]
    qkv = core[:, 2 * NHID:].reshape(b, NB, 3, MH, MDK)
    q2 = qkv[:, :, 0]
    k2 = qkv[:, :, 1]
    v2 = qkv[:, :, 2]
    s2 = jnp.einsum('bnhd,bmhd->bhnm', q2, k2) / float(np.sqrt(MDK))
    a2 = jax.nn.softmax(s2, axis=-1)
    o2 = jnp.einsum('bhnm,bmhd->bnhd', a2, v2).reshape(b, NB, MH * MDK)
    att_update = (jax.nn.sigmoid(o2 @ mg_w + mg_b)
                  * jnp.tanh(o2 @ mf_w + mf_b))
    h_new = hn.reshape(b, NB, BS)
    c_new = cn.reshape(b, NB, BS)
    h_new2 = h_new + att_update
    mb = mask[:, :, None]
    h_b = hx.reshape(b, NB, BS)
    c_b = cx.reshape(b, NB, BS)
    hx_out = (mb * h_new2 + (1.0 - mb) * h_b).reshape(b, NHID)
    cx_out = (mb * c_new + (1.0 - mb) * c_b).reshape(b, NHID)
    return (hx_out, cx_out), hx_out

  (_, _), outs = jax.lax.scan(step, (h0, c0), x_r)
  return outs


_MT = 448
_NT = 2048


def _decode_kernel(x_ref, w_ref, b_ref, out_ref):
  x = x_ref[...].astype(jnp.bfloat16)
  w = w_ref[...].astype(jnp.bfloat16)
  acc = jnp.dot(x, w, preferred_element_type=jnp.float32)
  out_ref[...] = acc + b_ref[...]


def _decode(outs2d, dec_w, dec_b):
  grid = (pl.cdiv(NTOKEN, _NT), NTOT // _MT)
  return pl.pallas_call(
      _decode_kernel,
      grid=grid,
      in_specs=[
          pl.BlockSpec((_MT, NHID), lambda n, m: (m, 0)),
          pl.BlockSpec((NHID, _NT), lambda n, m: (0, n)),
          pl.BlockSpec((1, _NT), lambda n, m: (0, n)),
      ],
      out_specs=pl.BlockSpec((_MT, _NT), lambda n, m: (m, n)),
      out_shape=jax.ShapeDtypeStruct((NTOT, NTOKEN), jnp.float32),
  )(outs2d, dec_w, dec_b.reshape(1, NTOKEN))


def kernel(input, hidden_h0, hidden_c0, emb, iq_w, iq_b, ik_w, ik_b, iv_w,
           iv_b, lstm_wih, lstm_whh, lstm_b, mq_w, mq_b, mk_w, mk_b, mv_w,
           mv_b, mf_w, mf_b, mg_w, mg_b, dec_w, dec_b):
  table_pad = jnp.pad(emb, ((0, 0), (0, DPAD - NINP)))
  idx = input.reshape(-1).astype(jnp.int32)
  idx_pad = jnp.pad(idx, (0, BPAD - NTOT))
  x_rows = _sc_gather(table_pad, idx_pad)
  x_r = x_rows[:NTOT, :NINP].reshape(SEQ, BATCH, NINP)

  outs = _recurrence(x_r, hidden_h0, hidden_c0, iq_w, iq_b, ik_w, ik_b,
                     iv_w, iv_b, lstm_wih, lstm_whh, lstm_b, mq_w, mq_b,
                     mk_w, mk_b, mv_w, mv_b, mf_w, mf_b, mg_w, mg_b)

  logits = _decode(outs.reshape(NTOT, NHID), dec_w, dec_b)
  return logits.reshape(SEQ, BATCH, NTOKEN)
